# Initial kernel scaffold; baseline (speedup 1.0000x reference)
import jax
import jax.numpy as jnp
from jax.experimental import pallas as pl


def kernel(feat, edge_index, W0, W1, W2, b2, L0, L1, L2, g0, be0, g1, be1):
    raise NotImplementedError("write your pallas kernel here")


# speedup vs baseline: 7.1938x; 7.1938x over previous
import functools

import jax
import jax.numpy as jnp
from jax import lax
from jax.experimental import pallas as pl
from jax.experimental.pallas import tpu as pltpu
from jax.experimental.pallas import tpu_sc as plsc

N = 10000
E = 320000
D = 128
H = 128
C = 40

NC = 2
NS = 16
NW = NC * NS
EW = E // NW
CH = 128
NFULL = EW // CH
TAIL = EW - NFULL * CH
NP = 10240
RPT = NP // NS
ZR = 128

BN = 2000
GRID = N // BN

_mesh = plsc.VectorSubcoreMesh(core_axis_name="c", subcore_axis_name="s")


@functools.partial(
    pl.kernel,
    mesh=_mesh,
    out_type=jax.ShapeDtypeStruct((NC, NP, H), jnp.float32),
    scratch_types=[
        pltpu.VMEM((CH,), jnp.int32),
        pltpu.VMEM((CH,), jnp.int32),
        pltpu.VMEM((TAIL,), jnp.int32),
        pltpu.VMEM((TAIL,), jnp.int32),
        pltpu.VMEM((CH, H), jnp.float32),
        pltpu.VMEM((CH, H), jnp.float32),
        pltpu.VMEM_SHARED((NP, H), jnp.float32),
        pltpu.SemaphoreType.DMA,
        pltpu.SemaphoreType.DMA,
        pltpu.SemaphoreType.DMA,
        pltpu.SemaphoreType.DMA,
    ],
)
def _count_sc(src_h, dst_h, e0_h, e1_h, zeros_h, out_h, src_v, dst_v,
              src_t, dst_t, e0_v, e1_v, acc, sem_a, sem_b, sem_c, sem_d):
    cid = lax.axis_index("c")
    sid = lax.axis_index("s")
    base = (sid * NC + cid) * EW

    pltpu.sync_copy(e0_h, e0_v)
    pltpu.sync_copy(e1_h, e1_v)
    pltpu.sync_copy(zeros_h, e0_v)
    for t in range(RPT // ZR):
        pltpu.sync_copy(e0_v, acc.at[pl.ds(sid * RPT + t * ZR, ZR)])
    pltpu.sync_copy(e0_h, e0_v)
    plsc.subcore_barrier()

    def body(j, _):
        c1 = pltpu.make_async_copy(src_h.at[pl.ds(base + j * CH, CH)], src_v,
                                   sem_a)
        c2 = pltpu.make_async_copy(dst_h.at[pl.ds(base + j * CH, CH)], dst_v,
                                   sem_b)
        c1.start()
        c2.start()
        c1.wait()
        c2.wait()
        a1 = pltpu.make_async_copy(e0_v, acc.at[dst_v], sem_c)
        a2 = pltpu.make_async_copy(e1_v, acc.at[src_v], sem_d)
        a1.start(add=True)
        a2.start(add=True)
        a1.wait()
        a2.wait()
        return 0

    lax.fori_loop(0, NFULL, body, 0)
    pltpu.sync_copy(src_h.at[pl.ds(base + NFULL * CH, TAIL)], src_t)
    pltpu.sync_copy(dst_h.at[pl.ds(base + NFULL * CH, TAIL)], dst_t)
    pltpu.sync_copy(e0_v.at[pl.ds(0, TAIL)], acc.at[dst_t], add=True)
    pltpu.sync_copy(e1_v.at[pl.ds(0, TAIL)], acc.at[src_t], add=True)
    plsc.subcore_barrier()
    for t in range(RPT // ZR):
        pltpu.sync_copy(acc.at[pl.ds(sid * RPT + t * ZR, ZR)], e0_v)
        pltpu.sync_copy(e0_v, out_h.at[cid, pl.ds(sid * RPT + t * ZR, ZR)])


def _make_segsum(W):
    @functools.partial(
        pl.kernel,
        mesh=_mesh,
        out_type=jax.ShapeDtypeStruct((NC, NP, W), jnp.float32),
        scratch_types=[
            pltpu.VMEM((CH,), jnp.int32),
            pltpu.VMEM((CH,), jnp.int32),
            pltpu.VMEM((CH,), jnp.int32),
            pltpu.VMEM((CH,), jnp.int32),
            pltpu.VMEM((TAIL,), jnp.int32),
            pltpu.VMEM((TAIL,), jnp.int32),
            pltpu.VMEM((CH, W), jnp.float32),
            pltpu.VMEM((CH, W), jnp.float32),
            pltpu.VMEM((TAIL, W), jnp.float32),
            pltpu.VMEM_SHARED((NP, W), jnp.float32),
            pltpu.SemaphoreType.DMA,
            pltpu.SemaphoreType.DMA,
            pltpu.SemaphoreType.DMA,
            pltpu.SemaphoreType.DMA,
            pltpu.SemaphoreType.DMA,
            pltpu.SemaphoreType.DMA,
        ],
    )
    def k(hn_h, src_h, dst_h, zeros_h, out_h,
          src0, src1, dst0, dst1, src_t, dst_t,
          rows0, rows1, rows_t, acc, sem0, sem1, semt, sem_i0, sem_i1,
          sem_w):
        cid = lax.axis_index("c")
        sid = lax.axis_index("s")
        base = (sid * NC + cid) * EW
        srcs = (src0, src1)
        dsts = (dst0, dst1)
        rows = (rows0, rows1)
        sems = (sem0, sem1)

        pltpu.sync_copy(zeros_h, rows0)
        for t in range(RPT // ZR):
            pltpu.sync_copy(rows0, acc.at[pl.ds(sid * RPT + t * ZR, ZR)])
        plsc.subcore_barrier()

        for s in range(2):
            pltpu.sync_copy(src_h.at[pl.ds(base + s * CH, CH)], srcs[s])
            pltpu.sync_copy(dst_h.at[pl.ds(base + s * CH, CH)], dsts[s])
            pltpu.async_copy(hn_h.at[srcs[s]], rows[s], sems[s])

        isems = (sem_i0, sem_i1)

        def body(i, _):
            for s in range(2):
                j = 2 * i + s
                pltpu.make_async_copy(hn_h.at[srcs[s]], rows[s], sems[s]).wait()
                w1 = pltpu.make_async_copy(rows[s], acc.at[dsts[s]], sem_w)
                w1.start(add=True)
                w1.wait()

                @pl.when(j + 2 < NFULL)
                def _():
                    c1 = pltpu.make_async_copy(
                        src_h.at[pl.ds(base + (j + 2) * CH, CH)], srcs[s],
                        isems[s])
                    c2 = pltpu.make_async_copy(
                        dst_h.at[pl.ds(base + (j + 2) * CH, CH)], dsts[s],
                        sem_w)
                    c1.start()
                    c2.start()
                    c1.wait()
                    c2.wait()
                    pltpu.async_copy(hn_h.at[srcs[s]], rows[s], sems[s])

            return 0

        lax.fori_loop(0, NFULL // 2, body, 0)
        pltpu.sync_copy(src_h.at[pl.ds(base + NFULL * CH, TAIL)], src_t)
        pltpu.sync_copy(dst_h.at[pl.ds(base + NFULL * CH, TAIL)], dst_t)
        pltpu.async_copy(hn_h.at[src_t], rows_t, semt).wait()
        pltpu.sync_copy(rows_t, acc.at[dst_t], add=True)
        plsc.subcore_barrier()
        for t in range(RPT // ZR):
            pltpu.sync_copy(acc.at[pl.ds(sid * RPT + t * ZR, ZR)], rows0)
            pltpu.sync_copy(rows0,
                            out_h.at[cid, pl.ds(sid * RPT + t * ZR, ZR)])

    return k


_segsum_h = _make_segsum(H)


def _ns_from(cnt_ref, kind):
    lane = 1 - kind
    c = cnt_ref[0] + cnt_ref[1]
    return lax.rsqrt(jnp.maximum(c[:, lane:lane + 1], 1.0))


def _pre0_body(cnt_ref, feat_ref, hn_ref):
    hn_ref[...] = feat_ref[...] * _ns_from(cnt_ref, 0)


def _postA_body(cnt_ref, agg_ref, h_ref, w_ref, l_ref, t_ref, st_ref):
    i = pl.program_id(0)
    a = (agg_ref[0] + agg_ref[1]) * _ns_from(cnt_ref, 1)
    t = (jnp.dot(a, w_ref[...], preferred_element_type=jnp.float32)
         + jnp.dot(h_ref[...], l_ref[...], preferred_element_type=jnp.float32))
    t_ref[...] = t
    s = jnp.sum(t, axis=0, keepdims=True)
    q = jnp.sum(t * t, axis=0, keepdims=True)

    @pl.when(i == 0)
    def _():
        st_ref[0] = s
        st_ref[1] = q

    @pl.when(i > 0)
    def _():
        st_ref[0] += s
        st_ref[1] += q


def _bn_relu(st_ref, t_ref, g_ref, b_ref):
    m = st_ref[0] / N
    v = st_ref[1] / N - m * m
    return jnp.maximum(
        (t_ref[...] - m) * lax.rsqrt(v + 1e-5) * g_ref[...] + b_ref[...], 0.0)


def _postB0_body(st_ref, cnt_ref, t_ref, g_ref, b_ref, h_ref, hn_ref):
    hh = _bn_relu(st_ref, t_ref, g_ref, b_ref)
    h_ref[...] = hh
    hn_ref[...] = hh * _ns_from(cnt_ref, 0)


def _final_body(cnt_ref, agg_ref, h_ref, w2_ref, l2_ref, b2_ref, out_ref):
    a = (agg_ref[0] + agg_ref[1]) * _ns_from(cnt_ref, 1)
    out_ref[...] = (jnp.dot(a, w2_ref[...], preferred_element_type=jnp.float32)
                    + b2_ref[...]
                    + jnp.dot(h_ref[...], l2_ref[...],
                              preferred_element_type=jnp.float32))


_cnt_spec = pl.BlockSpec((NC, BN, H), lambda i: (0, i, 0))
_row_spec = pl.BlockSpec((BN, D), lambda i: (i, 0))
_agg_spec = pl.BlockSpec((NC, BN, D), lambda i: (0, i, 0))
_st_spec = pl.BlockSpec((2, 1, H), lambda i: (0, 0, 0))
_w_spec = pl.BlockSpec((D, H), lambda i: (0, 0))
_g_spec = pl.BlockSpec((1, H), lambda i: (0, 0))


def _pre0(cnt, feat):
    return pl.pallas_call(
        _pre0_body,
        grid=(GRID,),
        in_specs=[_cnt_spec, _row_spec],
        out_specs=_row_spec,
        out_shape=jax.ShapeDtypeStruct((N, D), jnp.float32),
    )(cnt, feat)


def _postA(cnt, agg, h, w, l):
    return pl.pallas_call(
        _postA_body,
        grid=(GRID,),
        in_specs=[_cnt_spec, _agg_spec, _row_spec, _w_spec, _w_spec],
        out_specs=[_row_spec, _st_spec],
        out_shape=[jax.ShapeDtypeStruct((N, H), jnp.float32),
                   jax.ShapeDtypeStruct((2, 1, H), jnp.float32)],
    )(cnt, agg, h, w, l)


def _postB0(st, cnt, t, g, b):
    return pl.pallas_call(
        _postB0_body,
        grid=(GRID,),
        in_specs=[_st_spec, _cnt_spec, _row_spec, _g_spec, _g_spec],
        out_specs=[_row_spec, _row_spec],
        out_shape=[jax.ShapeDtypeStruct((N, H), jnp.float32),
                   jax.ShapeDtypeStruct((N, H), jnp.float32)],
    )(st, cnt, t, g, b)


def _final(cnt, agg, h, w2, l2, b2):
    return pl.pallas_call(
        _final_body,
        grid=(GRID,),
        in_specs=[_cnt_spec, _agg_spec, _row_spec,
                  pl.BlockSpec((D, C), lambda i: (0, 0)),
                  pl.BlockSpec((D, C), lambda i: (0, 0)),
                  pl.BlockSpec((1, C), lambda i: (0, 0))],
        out_specs=pl.BlockSpec((BN, C), lambda i: (i, 0)),
        out_shape=jax.ShapeDtypeStruct((N, C), jnp.float32),
    )(cnt, agg, h, w2, l2, b2)


def kernel(feat, edge_index, W0, W1, W2, b2, L0, L1, L2, g0, be0, g1, be1):
    src = edge_index[0]
    dst = edge_index[1]
    g0r = g0.reshape(1, H)
    be0r = be0.reshape(1, H)
    g1r = g1.reshape(1, H)
    be1r = be1.reshape(1, H)
    b2r = b2.reshape(1, C)

    zerosH = jnp.zeros((CH, H), jnp.float32)
    e0 = zerosH.at[:, 0].set(1.0)
    e1 = zerosH.at[:, 1].set(1.0)
    cnt = _count_sc(src, dst, e0, e1, zerosH)
    hn0 = _pre0(cnt, feat)
    agg0 = _segsum_h(hn0, src, dst, zerosH)
    t0, st0 = _postA(cnt, agg0, feat, W0, L0)
    h1, hn1 = _postB0(st0, cnt, t0, g0r, be0r)
    agg1 = _segsum_h(hn1, src, dst, zerosH)
    t1, st1 = _postA(cnt, agg1, h1, W1, L1)
    h2, hn2 = _postB0(st1, cnt, t1, g1r, be1r)
    agg2 = _segsum_h(hn2, src, dst, zerosH)
    return _final(cnt, agg2, h2, W2, L2, b2r)


# speedup vs baseline: 7.9535x; 1.1056x over previous
import functools

import jax
import jax.numpy as jnp
from jax import lax
from jax.experimental import pallas as pl
from jax.experimental.pallas import tpu as pltpu
from jax.experimental.pallas import tpu_sc as plsc

N = 10000
E = 320000
D = 128
H = 128
C = 40

NC = 2
NS = 16
NW = NC * NS
EW = E // NW
CH = 128
EWP = 10240
NFULLP = EWP // CH
BLK = 40
NP = 10240
RPT = NP // NS
ZR = 128

BN = 2000
GRID = N // BN

_mesh = plsc.VectorSubcoreMesh(core_axis_name="c", subcore_axis_name="s")


@functools.partial(
    pl.kernel,
    mesh=_mesh,
    out_type=jax.ShapeDtypeStruct((NC, NP, H), jnp.float32),
    scratch_types=[
        pltpu.VMEM((BLK, CH), jnp.int32),
        pltpu.VMEM((BLK, CH), jnp.int32),
        pltpu.VMEM((CH, H), jnp.float32),
        pltpu.VMEM((CH, H), jnp.float32),
        pltpu.VMEM_SHARED((NP, H), jnp.float32),
        pltpu.SemaphoreType.DMA,
        pltpu.SemaphoreType.DMA,
    ],
)
def _count_sc(src3_h, dst3_h, e0_h, e1_h, zeros_h, out_h,
              src_all, dst_all, e0_v, e1_v, acc, sem_c, sem_d):
    cid = lax.axis_index("c")
    sid = lax.axis_index("s")
    wid = sid * NC + cid

    pltpu.sync_copy(zeros_h, e0_v)
    for t in range(RPT // ZR):
        pltpu.sync_copy(e0_v, acc.at[pl.ds(sid * RPT + t * ZR, ZR)])
    pltpu.sync_copy(e0_h, e0_v)
    pltpu.sync_copy(e1_h, e1_v)
    plsc.subcore_barrier()

    for b in range(NFULLP // BLK):
        pltpu.sync_copy(src3_h.at[wid, pl.ds(b * BLK, BLK)], src_all)
        pltpu.sync_copy(dst3_h.at[wid, pl.ds(b * BLK, BLK)], dst_all)

        def body(j, _):
            a1 = pltpu.make_async_copy(e0_v, acc.at[dst_all.at[j]], sem_c)
            a2 = pltpu.make_async_copy(e1_v, acc.at[src_all.at[j]], sem_d)
            a1.start(add=True)
            a2.start(add=True)
            a1.wait()
            a2.wait()
            return 0

        lax.fori_loop(0, BLK, body, 0)
    plsc.subcore_barrier()
    for t in range(RPT // ZR):
        pltpu.sync_copy(acc.at[pl.ds(sid * RPT + t * ZR, ZR)], e0_v)
        pltpu.sync_copy(e0_v, out_h.at[cid, pl.ds(sid * RPT + t * ZR, ZR)])


def _make_segsum(W):
    @functools.partial(
        pl.kernel,
        mesh=_mesh,
        out_type=jax.ShapeDtypeStruct((NC, NP, W), jnp.float32),
        scratch_types=[
            pltpu.VMEM((BLK, CH), jnp.int32),
            pltpu.VMEM((BLK, CH), jnp.int32),
            pltpu.VMEM((CH, W), jnp.float32),
            pltpu.VMEM((CH, W), jnp.float32),
            pltpu.VMEM_SHARED((NP, W), jnp.float32),
            pltpu.SemaphoreType.DMA,
            pltpu.SemaphoreType.DMA,
            pltpu.SemaphoreType.DMA,
        ],
    )
    def k(hn_h, src3_h, dst3_h, zeros_h, out_h,
          src_all, dst_all, rows0, rows1, acc, sem0, sem1, sem_w):
        cid = lax.axis_index("c")
        sid = lax.axis_index("s")
        wid = sid * NC + cid
        rows = (rows0, rows1)
        sems = (sem0, sem1)

        pltpu.sync_copy(zeros_h, rows0)
        for t in range(RPT // ZR):
            pltpu.sync_copy(rows0, acc.at[pl.ds(sid * RPT + t * ZR, ZR)])
        plsc.subcore_barrier()

        for b in range(NFULLP // BLK):
            pltpu.sync_copy(src3_h.at[wid, pl.ds(b * BLK, BLK)], src_all)
            pltpu.sync_copy(dst3_h.at[wid, pl.ds(b * BLK, BLK)], dst_all)
            for s in range(2):
                pltpu.async_copy(hn_h.at[src_all.at[s]], rows[s], sems[s])

            def body(i, _):
                for s in range(2):
                    q = 2 * i + s
                    pltpu.make_async_copy(hn_h.at[src_all.at[q]], rows[s],
                                          sems[s]).wait()
                    w1 = pltpu.make_async_copy(rows[s], acc.at[dst_all.at[q]],
                                               sem_w)
                    w1.start(add=True)
                    w1.wait()

                    @pl.when(q + 2 < BLK)
                    def _():
                        pltpu.async_copy(hn_h.at[src_all.at[q + 2]], rows[s],
                                         sems[s])

                return 0

            lax.fori_loop(0, BLK // 2, body, 0)
        plsc.subcore_barrier()
        for t in range(RPT // ZR):
            pltpu.sync_copy(acc.at[pl.ds(sid * RPT + t * ZR, ZR)], rows0)
            pltpu.sync_copy(rows0,
                            out_h.at[cid, pl.ds(sid * RPT + t * ZR, ZR)])

    return k


_segsum_h = _make_segsum(H)


def _ns_from(cnt_ref, kind):
    lane = 1 - kind
    c = cnt_ref[0] + cnt_ref[1]
    return lax.rsqrt(jnp.maximum(c[:, lane:lane + 1], 1.0))


def _pre0_body(cnt_ref, feat_ref, hn_ref):
    hn_ref[...] = feat_ref[...] * _ns_from(cnt_ref, 0)


def _postA_body(cnt_ref, agg_ref, h_ref, w_ref, l_ref, t_ref, st_ref):
    i = pl.program_id(0)
    a = (agg_ref[0] + agg_ref[1]) * _ns_from(cnt_ref, 1)
    t = (jnp.dot(a, w_ref[...], preferred_element_type=jnp.float32)
         + jnp.dot(h_ref[...], l_ref[...], preferred_element_type=jnp.float32))
    t_ref[...] = t
    s = jnp.sum(t, axis=0, keepdims=True)
    q = jnp.sum(t * t, axis=0, keepdims=True)

    @pl.when(i == 0)
    def _():
        st_ref[0] = s
        st_ref[1] = q

    @pl.when(i > 0)
    def _():
        st_ref[0] += s
        st_ref[1] += q


def _bn_relu(st_ref, t_ref, g_ref, b_ref):
    m = st_ref[0] / N
    v = st_ref[1] / N - m * m
    return jnp.maximum(
        (t_ref[...] - m) * lax.rsqrt(v + 1e-5) * g_ref[...] + b_ref[...], 0.0)


def _postB0_body(st_ref, cnt_ref, t_ref, g_ref, b_ref, h_ref, hn_ref):
    hh = _bn_relu(st_ref, t_ref, g_ref, b_ref)
    h_ref[...] = hh
    hn_ref[...] = hh * _ns_from(cnt_ref, 0)


def _final_body(cnt_ref, agg_ref, h_ref, w2_ref, l2_ref, b2_ref, out_ref):
    a = (agg_ref[0] + agg_ref[1]) * _ns_from(cnt_ref, 1)
    out_ref[...] = (jnp.dot(a, w2_ref[...], preferred_element_type=jnp.float32)
                    + b2_ref[...]
                    + jnp.dot(h_ref[...], l2_ref[...],
                              preferred_element_type=jnp.float32))


_cnt_spec = pl.BlockSpec((NC, BN, H), lambda i: (0, i, 0))
_row_spec = pl.BlockSpec((BN, D), lambda i: (i, 0))
_agg_spec = pl.BlockSpec((NC, BN, D), lambda i: (0, i, 0))
_st_spec = pl.BlockSpec((2, 1, H), lambda i: (0, 0, 0))
_w_spec = pl.BlockSpec((D, H), lambda i: (0, 0))
_g_spec = pl.BlockSpec((1, H), lambda i: (0, 0))


def _pre0(cnt, feat):
    return pl.pallas_call(
        _pre0_body,
        grid=(GRID,),
        in_specs=[_cnt_spec, _row_spec],
        out_specs=_row_spec,
        out_shape=jax.ShapeDtypeStruct((N, D), jnp.float32),
    )(cnt, feat)


def _postA(cnt, agg, h, w, l):
    return pl.pallas_call(
        _postA_body,
        grid=(GRID,),
        in_specs=[_cnt_spec, _agg_spec, _row_spec, _w_spec, _w_spec],
        out_specs=[_row_spec, _st_spec],
        out_shape=[jax.ShapeDtypeStruct((N, H), jnp.float32),
                   jax.ShapeDtypeStruct((2, 1, H), jnp.float32)],
    )(cnt, agg, h, w, l)


def _postB0(st, cnt, t, g, b):
    return pl.pallas_call(
        _postB0_body,
        grid=(GRID,),
        in_specs=[_st_spec, _cnt_spec, _row_spec, _g_spec, _g_spec],
        out_specs=[_row_spec, _row_spec],
        out_shape=[jax.ShapeDtypeStruct((N, H), jnp.float32),
                   jax.ShapeDtypeStruct((N, H), jnp.float32)],
    )(st, cnt, t, g, b)


def _final(cnt, agg, h, w2, l2, b2):
    return pl.pallas_call(
        _final_body,
        grid=(GRID,),
        in_specs=[_cnt_spec, _agg_spec, _row_spec,
                  pl.BlockSpec((D, C), lambda i: (0, 0)),
                  pl.BlockSpec((D, C), lambda i: (0, 0)),
                  pl.BlockSpec((1, C), lambda i: (0, 0))],
        out_specs=pl.BlockSpec((BN, C), lambda i: (i, 0)),
        out_shape=jax.ShapeDtypeStruct((N, C), jnp.float32),
    )(cnt, agg, h, w2, l2, b2)


def kernel(feat, edge_index, W0, W1, W2, b2, L0, L1, L2, g0, be0, g1, be1):
    src = edge_index[0]
    dst = edge_index[1]
    g0r = g0.reshape(1, H)
    be0r = be0.reshape(1, H)
    g1r = g1.reshape(1, H)
    be1r = be1.reshape(1, H)
    b2r = b2.reshape(1, C)

    zerosH = jnp.zeros((CH, H), jnp.float32)
    e0 = zerosH.at[:, 0].set(1.0)
    e1 = zerosH.at[:, 1].set(1.0)
    npad = NW * EWP - E
    pad_ar = jnp.arange(npad, dtype=jnp.int32)
    srcp = jnp.concatenate([src, pad_ar % N])
    dstp = jnp.concatenate([dst, N + pad_ar % (NP - N)])
    srcc = jnp.concatenate([src, N + pad_ar % (NP - N)])
    src3 = srcp.reshape(NW, NFULLP, CH)
    dst3 = dstp.reshape(NW, NFULLP, CH)
    src3c = srcc.reshape(NW, NFULLP, CH)
    cnt = _count_sc(src3c, dst3, e0, e1, zerosH)
    hn0 = _pre0(cnt, feat)
    agg0 = _segsum_h(hn0, src3, dst3, zerosH)
    t0, st0 = _postA(cnt, agg0, feat, W0, L0)
    h1, hn1 = _postB0(st0, cnt, t0, g0r, be0r)
    agg1 = _segsum_h(hn1, src3, dst3, zerosH)
    t1, st1 = _postA(cnt, agg1, h1, W1, L1)
    h2, hn2 = _postB0(st1, cnt, t1, g1r, be1r)
    agg2 = _segsum_h(hn2, src3, dst3, zerosH)
    return _final(cnt, agg2, h2, W2, L2, b2r)
